# baseline (speedup 1.0000x reference)
import jax
import jax.numpy as jnp
from jax.experimental import pallas as pl

_F32_MIN = float(jnp.finfo(jnp.float32).min)


def _mm_relu_body(a_ref, w_ref, o_ref):
    o_ref[...] = jnp.maximum(
        jnp.dot(a_ref[...], w_ref[...], preferred_element_type=jnp.float32), 0.0
    )


def _conv1_matmul(patches, w, bm=512):
    m, k = patches.shape
    n = w.shape[1]
    return pl.pallas_call(
        _mm_relu_body,
        grid=(m // bm,),
        in_specs=[
            pl.BlockSpec((bm, k), lambda i: (i, 0)),
            pl.BlockSpec((k, n), lambda i: (0, 0)),
        ],
        out_specs=pl.BlockSpec((bm, n), lambda i: (i, 0)),
        out_shape=jax.ShapeDtypeStruct((m, n), jnp.float32),
    )(patches, w)


def _mm_relu_pool_body(a_ref, w_ref, o_ref):
    h = jnp.maximum(
        jnp.dot(a_ref[...], w_ref[...], preferred_element_type=jnp.float32), 0.0
    )
    o_ref[...] = (jnp.sum(h, axis=0, keepdims=True)
                  * jnp.float32(1.0 / 784.0))[None]


def _conv2_pool(patches, w):
    m, k = patches.shape
    n = w.shape[1]
    return pl.pallas_call(
        _mm_relu_pool_body,
        grid=(m // 784,),
        in_specs=[
            pl.BlockSpec((784, k), lambda i: (i, 0)),
            pl.BlockSpec((k, n), lambda i: (0, 0)),
        ],
        out_specs=pl.BlockSpec((1, 1, n), lambda i: (i, 0, 0)),
        out_shape=jax.ShapeDtypeStruct((m // 784, 1, n), jnp.float32),
    )(patches, w).reshape(m // 784, n)


def _head_body(poolT_ref, fcw_ref, fcb_ref, g_ref, wvw_ref, wvb_ref,
               w1w_ref, w1b_ref, vw_ref, vb_ref, pw_ref,
               out0_ref, pred_ref, attn_ref, masked_ref):
    featT = jnp.dot(fcw_ref[...], poolT_ref[...],
                    preferred_element_type=jnp.float32) + fcb_ref[...]
    eps = jnp.float32(jnp.finfo(jnp.float32).eps)
    ms = jnp.mean(featT * featT, axis=0, keepdims=True)
    featT = featT * jax.lax.rsqrt(ms + eps) * g_ref[...]
    x_vT = jnp.dot(wvw_ref[...], featT,
                   preferred_element_type=jnp.float32) + wvb_ref[...]
    a1T = jnp.tanh(jnp.dot(w1w_ref[...], featT,
                           preferred_element_type=jnp.float32) + w1b_ref[...])
    raw_row = jnp.dot(vw_ref[...], a1T,
                      preferred_element_type=jnp.float32) + vb_ref[...]
    pred_row = jnp.dot(pw_ref[...], x_vT,
                       preferred_element_type=jnp.float32) * 100.0

    s = jnp.concatenate(
        [raw_row[:, 16 * c:16 * (c + 1)] for c in range(4)], axis=0)
    t = jax.lax.broadcasted_iota(jnp.int32, (4, 16), 1)
    neg = jnp.full((4, 16), jnp.float32(-3.0e38))
    processed = jnp.zeros((4, 16), dtype=jnp.float32)
    mask = jnp.ones((4, 16), dtype=jnp.float32)
    keep = jnp.ones((4, 16), dtype=jnp.float32)
    for _ in range(16):
        cand = jnp.where(processed > 0.5, neg, s)
        mx = jnp.max(cand, axis=-1, keepdims=True)
        ii = jnp.min(jnp.where(cand == mx, t, 99), axis=-1, keepdims=True)
        is_i = jnp.where(t == ii, 1.0, 0.0)
        cond = jnp.max(is_i * mask, axis=-1, keepdims=True)
        window = jnp.abs(t - ii) <= 1
        mask = jnp.where(cond > 0.5, jnp.where(window, is_i, mask), mask)
        keep = jnp.where(t == ii, jnp.where(cond > 0.5, keep, 0.0), keep)
        processed = jnp.maximum(processed, is_i)

    keep_row = jnp.concatenate(
        [keep[c:c + 1, :] for c in range(4)], axis=1)

    rawm = jnp.where(keep_row > 0.5, raw_row, _F32_MIN)
    fp = 1.0 / (1.0 + jnp.exp(-rawm))
    fp = jnp.where(jnp.abs(fp) < jnp.inf, fp, 0.0)

    v = jnp.maximum(fp, 0.0)
    i0 = jax.lax.broadcasted_iota(jnp.int32, (64, 64), 0)
    i1 = jax.lax.broadcasted_iota(jnp.int32, (64, 64), 1)
    eye = (i0 == i1).astype(jnp.float32)
    vcol = jax.lax.dot_general(eye, v, (((1,), (1,)), ((), ())),
                               preferred_element_type=jnp.float32)
    gi = jax.lax.broadcasted_iota(jnp.int32, (1, 64), 1)
    gj = jax.lax.broadcasted_iota(jnp.int32, (64, 1), 0)
    higher = vcol > v
    tie = vcol == v
    vj_b = jnp.broadcast_to(vcol, (64, 64))
    csum_part = jnp.where(higher, vj_b, jnp.where(tie & (gj <= gi), vj_b, 0.0))
    csum = jnp.sum(csum_part, axis=0, keepdims=True)
    rank_part = jnp.where(higher, 1.0, jnp.where(tie & (gj < gi), 1.0, 0.0))
    rank = jnp.sum(rank_part, axis=0, keepdims=True)
    total = jnp.sum(v, axis=1, keepdims=True)
    keep_tp = jnp.where(csum / (total + 1e-08) <= 0.7, 1.0,
                        jnp.where(rank < 3.0, 1.0, 0.0))

    masked = fp * keep_tp
    ssum = jnp.sum(masked, axis=1, keepdims=True)
    attn = masked / (ssum + 1e-08)
    attn = jnp.where(ssum <= 0.0, jnp.full((1, 64), jnp.float32(1.0 / 64.0)), attn)

    out0_ref[...] = jnp.sum(attn * pred_row, axis=1, keepdims=True)
    pred_ref[...] = pred_row
    attn_ref[...] = attn
    masked_ref[...] = masked


def _head(poolT, fc_w, fc_b, rms_g, WV_w, WV_b, W1_w, W1_b, V_w, V_b, P_w):
    out_shapes = (
        jax.ShapeDtypeStruct((1, 1), jnp.float32),
        jax.ShapeDtypeStruct((1, 64), jnp.float32),
        jax.ShapeDtypeStruct((1, 64), jnp.float32),
        jax.ShapeDtypeStruct((1, 64), jnp.float32),
    )
    return pl.pallas_call(_head_body, out_shape=out_shapes)(
        poolT, fc_w, fc_b.reshape(2048, 1), rms_g.reshape(2048, 1),
        WV_w, WV_b.reshape(128, 1), W1_w, W1_b.reshape(64, 1),
        V_w, V_b.reshape(1, 1), P_w)


def _conv1_patches(x):
    xi = x.reshape(64, 3, 224, 224)
    xp = jnp.pad(xi, ((0, 0), (0, 0), (1, 3), (1, 3)))
    xa = xp.reshape(64, 3, 57, 4, 57, 4).transpose(0, 2, 4, 3, 5, 1)
    xa = xa.reshape(64, 57, 57, 48)
    xb = jnp.concatenate([xa[:, 0:56], xa[:, 1:57]], axis=3)
    xc = jnp.concatenate([xb[:, :, 0:56], xb[:, :, 1:57]], axis=3)
    return xc.reshape(64 * 56 * 56, 192)


def _conv1_weight(conv1_w):
    wp = jnp.pad(conv1_w, ((0, 0), (0, 0), (0, 1), (0, 1)))
    wp = wp.reshape(64, 3, 2, 4, 2, 4)
    return wp.transpose(4, 2, 3, 5, 1, 0).reshape(192, 64)


def _conv2_patches(h1):
    h1r = h1.reshape(64, 56, 56, 64)
    h1p = jnp.pad(h1r, ((0, 0), (0, 2), (0, 2), (0, 0)))
    xs = h1p.reshape(64, 29, 2, 29, 2, 64).transpose(0, 1, 3, 2, 4, 5)
    xs = xs.reshape(64, 29, 29, 256)
    p1 = xs[:, 0:28, 0:28, :]
    p2a = xs[:, 0:28, 1:29, 0:64]
    p2b = xs[:, 0:28, 1:29, 128:192]
    p3 = xs[:, 1:29, 0:28, 0:128]
    p4 = xs[:, 1:29, 1:29, 0:64]
    patches = jnp.concatenate([p1, p2a, p2b, p3, p4], axis=3)
    return patches.reshape(64 * 28 * 28, 576)


def _conv2_weight(conv2_w):
    wt = conv2_w.transpose(2, 3, 1, 0)
    order = [(0, 0), (0, 1), (1, 0), (1, 1), (0, 2), (1, 2), (2, 0), (2, 1), (2, 2)]
    return jnp.concatenate([wt[kh, kw] for kh, kw in order], axis=0)


def kernel(x, conv1_w, conv2_w, fc_w, fc_b, rms_g, WV_w, WV_b, W1_w, W1_b, V_w, V_b, P_w):
    h1 = _conv1_matmul(_conv1_patches(x), _conv1_weight(conv1_w))
    pooled = _conv2_pool(_conv2_patches(h1), _conv2_weight(conv2_w))
    out0, pred_row, attn_row, masked_row = _head(
        pooled.T, fc_w, fc_b, rms_g, WV_w, WV_b, W1_w, W1_b, V_w, V_b, P_w)
    pred_by_frame = pred_row.reshape(1, 4, 16, 1)
    attn = attn_row.reshape(1, 4, 16, 1)
    masked = masked_row.reshape(1, 4, 16, 1)
    return out0, pred_by_frame, attn, masked


# speedup vs baseline: 2.4809x; 2.4809x over previous
import jax
import jax.numpy as jnp
from jax.experimental import pallas as pl

_F32_MIN = float(jnp.finfo(jnp.float32).min)


def _conv1_body(x_ref, w_ref, oee_ref, oeo_ref, ooe_ref, ooo_ref):
    xa = x_ref[0]
    acc = jnp.zeros((56, 56, 64), jnp.float32)
    for by in range(2):
        lhs = xa[by:56 + by].reshape(3192, 48)
        o = jnp.dot(lhs, w_ref[by], preferred_element_type=jnp.float32)
        o = o.reshape(56, 57, 128)
        acc = acc + o[:, 0:56, 0:64] + o[:, 1:57, 64:128]
    h = jnp.maximum(acc, 0.0)
    hp = h.reshape(28, 2, 28, 2, 64)
    for p, q, ref in ((0, 0, oee_ref), (0, 1, oeo_ref),
                      (1, 0, ooe_ref), (1, 1, ooo_ref)):
        plane = jnp.pad(hp[:, p, :, q, :], ((0, 1), (0, 1), (0, 0)))
        ref[...] = plane[None]


def _conv1(xs2d, w2):
    shp = jax.ShapeDtypeStruct((64, 29, 29, 64), jnp.float32)
    return pl.pallas_call(
        _conv1_body,
        grid=(64,),
        in_specs=[
            pl.BlockSpec((1, 57, 57, 48), lambda i: (i, 0, 0, 0)),
            pl.BlockSpec((2, 48, 128), lambda i: (0, 0, 0)),
        ],
        out_specs=[pl.BlockSpec((1, 29, 29, 64), lambda i: (i, 0, 0, 0))] * 4,
        out_shape=[shp] * 4,
    )(xs2d, w2)


def _conv2_body(hee_ref, heo_ref, hoe_ref, hoo_ref, wa_ref, wb_ref, o_ref):
    he = {}
    he[(0, 0)] = hee_ref[0]
    he[(0, 1)] = heo_ref[0]
    he[(1, 0)] = hoe_ref[0]
    he[(1, 1)] = hoo_ref[0]

    def piece(kh, kw):
        rp, dr = (kh % 2, 0) if kh < 2 else (0, 1)
        cp = kw % 2 if kw < 2 else 0
        return he[(rp, cp)][dr:28 + dr].reshape(812, 64)

    lhs_a = jnp.concatenate(
        [piece(kh, kw) for kh in range(3) for kw in (0, 1)], axis=1)
    lhs_b = jnp.concatenate([piece(kh, 2) for kh in range(3)], axis=1)
    oa = jnp.dot(lhs_a, wa_ref[...], preferred_element_type=jnp.float32)
    ob = jnp.dot(lhs_b, wb_ref[...], preferred_element_type=jnp.float32)
    oa = oa.reshape(28, 29, 128)
    ob = ob.reshape(28, 29, 128)
    h = jnp.maximum(oa[:, 0:28, :] + ob[:, 1:29, :], 0.0)
    pooled = jnp.sum(h, axis=(0, 1), keepdims=True) * jnp.float32(1.0 / 784.0)
    o_ref[...] = pooled


def _conv2_pool(hee, heo, hoe, hoo, wa, wb):
    hspec = pl.BlockSpec((1, 29, 29, 64), lambda i: (i, 0, 0, 0))
    return pl.pallas_call(
        _conv2_body,
        grid=(64,),
        in_specs=[hspec, hspec, hspec, hspec,
                  pl.BlockSpec((384, 128), lambda i: (0, 0)),
                  pl.BlockSpec((192, 128), lambda i: (0, 0))],
        out_specs=pl.BlockSpec((1, 1, 128), lambda i: (i, 0, 0)),
        out_shape=jax.ShapeDtypeStruct((64, 1, 128), jnp.float32),
    )(hee, heo, hoe, hoo, wa, wb).reshape(64, 128)


def _head_body(poolT_ref, fcw_ref, fcb_ref, g_ref, wvw_ref, wvb_ref,
               w1w_ref, w1b_ref, vw_ref, vb_ref, pw_ref,
               out0_ref, pred_ref, attn_ref, masked_ref):
    featT = jnp.dot(fcw_ref[...], poolT_ref[...],
                    preferred_element_type=jnp.float32) + fcb_ref[...]
    eps = jnp.float32(jnp.finfo(jnp.float32).eps)
    ms = jnp.mean(featT * featT, axis=0, keepdims=True)
    featT = featT * jax.lax.rsqrt(ms + eps) * g_ref[...]
    x_vT = jnp.dot(wvw_ref[...], featT,
                   preferred_element_type=jnp.float32) + wvb_ref[...]
    a1T = jnp.tanh(jnp.dot(w1w_ref[...], featT,
                           preferred_element_type=jnp.float32) + w1b_ref[...])
    raw_row = jnp.dot(vw_ref[...], a1T,
                      preferred_element_type=jnp.float32) + vb_ref[...]
    pred_row = jnp.dot(pw_ref[...], x_vT,
                       preferred_element_type=jnp.float32) * 100.0

    s = jnp.concatenate(
        [raw_row[:, 16 * c:16 * (c + 1)] for c in range(4)], axis=0)
    t = jax.lax.broadcasted_iota(jnp.int32, (4, 16), 1)
    neg = jnp.full((4, 16), jnp.float32(-3.0e38))
    processed = jnp.zeros((4, 16), dtype=jnp.float32)
    mask = jnp.ones((4, 16), dtype=jnp.float32)
    keep = jnp.ones((4, 16), dtype=jnp.float32)
    for _ in range(16):
        cand = jnp.where(processed > 0.5, neg, s)
        mx = jnp.max(cand, axis=-1, keepdims=True)
        ii = jnp.min(jnp.where(cand == mx, t, 99), axis=-1, keepdims=True)
        is_i = jnp.where(t == ii, 1.0, 0.0)
        cond = jnp.max(is_i * mask, axis=-1, keepdims=True)
        window = jnp.abs(t - ii) <= 1
        mask = jnp.where(cond > 0.5, jnp.where(window, is_i, mask), mask)
        keep = jnp.where(t == ii, jnp.where(cond > 0.5, keep, 0.0), keep)
        processed = jnp.maximum(processed, is_i)

    keep_row = jnp.concatenate(
        [keep[c:c + 1, :] for c in range(4)], axis=1)

    rawm = jnp.where(keep_row > 0.5, raw_row, _F32_MIN)
    fp = 1.0 / (1.0 + jnp.exp(-rawm))
    fp = jnp.where(jnp.abs(fp) < jnp.inf, fp, 0.0)

    v = jnp.maximum(fp, 0.0)
    i0 = jax.lax.broadcasted_iota(jnp.int32, (64, 64), 0)
    i1 = jax.lax.broadcasted_iota(jnp.int32, (64, 64), 1)
    eye = (i0 == i1).astype(jnp.float32)
    vcol = jax.lax.dot_general(eye, v, (((1,), (1,)), ((), ())),
                               preferred_element_type=jnp.float32)
    gi = jax.lax.broadcasted_iota(jnp.int32, (1, 64), 1)
    gj = jax.lax.broadcasted_iota(jnp.int32, (64, 1), 0)
    higher = vcol > v
    tie = vcol == v
    vj_b = jnp.broadcast_to(vcol, (64, 64))
    csum_part = jnp.where(higher, vj_b, jnp.where(tie & (gj <= gi), vj_b, 0.0))
    csum = jnp.sum(csum_part, axis=0, keepdims=True)
    rank_part = jnp.where(higher, 1.0, jnp.where(tie & (gj < gi), 1.0, 0.0))
    rank = jnp.sum(rank_part, axis=0, keepdims=True)
    total = jnp.sum(v, axis=1, keepdims=True)
    keep_tp = jnp.where(csum / (total + 1e-08) <= 0.7, 1.0,
                        jnp.where(rank < 3.0, 1.0, 0.0))

    masked = fp * keep_tp
    ssum = jnp.sum(masked, axis=1, keepdims=True)
    attn = masked / (ssum + 1e-08)
    attn = jnp.where(ssum <= 0.0, jnp.full((1, 64), jnp.float32(1.0 / 64.0)), attn)

    out0_ref[...] = jnp.sum(attn * pred_row, axis=1, keepdims=True)
    pred_ref[...] = pred_row
    attn_ref[...] = attn
    masked_ref[...] = masked


def _head(poolT, fc_w, fc_b, rms_g, WV_w, WV_b, W1_w, W1_b, V_w, V_b, P_w):
    out_shapes = (
        jax.ShapeDtypeStruct((1, 1), jnp.float32),
        jax.ShapeDtypeStruct((1, 64), jnp.float32),
        jax.ShapeDtypeStruct((1, 64), jnp.float32),
        jax.ShapeDtypeStruct((1, 64), jnp.float32),
    )
    return pl.pallas_call(_head_body, out_shape=out_shapes)(
        poolT, fc_w, fc_b.reshape(2048, 1), rms_g.reshape(2048, 1),
        WV_w, WV_b.reshape(128, 1), W1_w, W1_b.reshape(64, 1),
        V_w, V_b.reshape(1, 1), P_w)


def _conv1_s2d(x):
    xi = x.reshape(64, 3, 224, 224)
    xp = jnp.pad(xi, ((0, 0), (0, 0), (1, 3), (1, 3)))
    xa = xp.reshape(64, 3, 57, 4, 57, 4).transpose(0, 2, 4, 3, 5, 1)
    return xa.reshape(64, 57, 57, 48)


def _conv1_weight(conv1_w):
    wp = jnp.pad(conv1_w, ((0, 0), (0, 0), (0, 1), (0, 1)))
    wp = wp.reshape(64, 3, 2, 4, 2, 4)
    return wp.transpose(2, 3, 5, 1, 4, 0).reshape(2, 48, 128)


def _conv2_weights(conv2_w):
    wt = conv2_w.transpose(2, 3, 1, 0)
    wa = jnp.concatenate(
        [wt[kh, kw] for kh in range(3) for kw in (0, 1)], axis=0)
    wb = jnp.concatenate([wt[kh, 2] for kh in range(3)], axis=0)
    return wa, wb


def kernel(x, conv1_w, conv2_w, fc_w, fc_b, rms_g, WV_w, WV_b, W1_w, W1_b, V_w, V_b, P_w):
    hee, heo, hoe, hoo = _conv1(_conv1_s2d(x), _conv1_weight(conv1_w))
    wa, wb = _conv2_weights(conv2_w)
    pooled = _conv2_pool(hee, heo, hoe, hoo, wa, wb)
    out0, pred_row, attn_row, masked_row = _head(
        pooled.T, fc_w, fc_b, rms_g, WV_w, WV_b, W1_w, W1_b, V_w, V_b, P_w)
    pred_by_frame = pred_row.reshape(1, 4, 16, 1)
    attn = attn_row.reshape(1, 4, 16, 1)
    masked = masked_row.reshape(1, 4, 16, 1)
    return out0, pred_by_frame, attn, masked


# speedup vs baseline: 2.9473x; 1.1880x over previous
import jax
import jax.numpy as jnp
from jax.experimental import pallas as pl

_F32_MIN = float(jnp.finfo(jnp.float32).min)


def _conv1_body(x_ref, w_ref, oee_ref, oeo_ref, ooe_ref, ooo_ref):
    xa = x_ref[0]
    acc = jnp.zeros((3192, 128), jnp.float32)
    for by in range(2):
        lhs = xa[57 * by:57 * by + 3192, :]
        acc = acc + jnp.dot(lhs, w_ref[by], preferred_element_type=jnp.float32)
    a3 = acc.reshape(56, 57, 128)
    h = jnp.maximum(a3[:, 0:56, 0:64] + a3[:, 1:57, 64:128], 0.0)
    hp = h.reshape(28, 2, 28, 2, 64)
    for p, q, ref in ((0, 0, oee_ref), (0, 1, oeo_ref),
                      (1, 0, ooe_ref), (1, 1, ooo_ref)):
        plane = jnp.pad(hp[:, p, :, q, :], ((0, 1), (0, 1), (0, 0)))
        ref[...] = plane[None]


def _conv1(xs2d, w2):
    shp = jax.ShapeDtypeStruct((64, 29, 29, 64), jnp.float32)
    return pl.pallas_call(
        _conv1_body,
        grid=(64,),
        in_specs=[
            pl.BlockSpec((1, 3249, 48), lambda i: (i, 0, 0)),
            pl.BlockSpec((2, 48, 128), lambda i: (0, 0, 0)),
        ],
        out_specs=[pl.BlockSpec((1, 29, 29, 64), lambda i: (i, 0, 0, 0))] * 4,
        out_shape=[shp] * 4,
    )(xs2d, w2)


def _conv2_body(hee_ref, heo_ref, hoe_ref, hoo_ref, wa_ref, wb_ref, o_ref):
    he = {}
    he[(0, 0)] = hee_ref[0]
    he[(0, 1)] = heo_ref[0]
    he[(1, 0)] = hoe_ref[0]
    he[(1, 1)] = hoo_ref[0]

    def piece(kh, kw):
        rp, dr = (kh % 2, 0) if kh < 2 else (0, 1)
        cp = kw % 2 if kw < 2 else 0
        return he[(rp, cp)][dr:28 + dr].reshape(812, 64)

    lhs_a = jnp.concatenate(
        [piece(kh, kw) for kh in range(3) for kw in (0, 1)], axis=1)
    lhs_b = jnp.concatenate([piece(kh, 2) for kh in range(3)], axis=1)
    oa = jnp.dot(lhs_a, wa_ref[...], preferred_element_type=jnp.float32)
    ob = jnp.dot(lhs_b, wb_ref[...], preferred_element_type=jnp.float32)
    oa = oa.reshape(28, 29, 128)
    ob = ob.reshape(28, 29, 128)
    h = jnp.maximum(oa[:, 0:28, :] + ob[:, 1:29, :], 0.0)
    pooled = jnp.sum(h, axis=(0, 1), keepdims=True) * jnp.float32(1.0 / 784.0)
    o_ref[...] = pooled


def _conv2_pool(hee, heo, hoe, hoo, wa, wb):
    hspec = pl.BlockSpec((1, 29, 29, 64), lambda i: (i, 0, 0, 0))
    return pl.pallas_call(
        _conv2_body,
        grid=(64,),
        in_specs=[hspec, hspec, hspec, hspec,
                  pl.BlockSpec((384, 128), lambda i: (0, 0)),
                  pl.BlockSpec((192, 128), lambda i: (0, 0))],
        out_specs=pl.BlockSpec((1, 1, 128), lambda i: (i, 0, 0)),
        out_shape=jax.ShapeDtypeStruct((64, 1, 128), jnp.float32),
    )(hee, heo, hoe, hoo, wa, wb).reshape(64, 128)


def _head_body(poolT_ref, fcw_ref, fcb_ref, g_ref, wvw_ref, wvb_ref,
               w1w_ref, w1b_ref, vw_ref, vb_ref, pw_ref,
               out0_ref, pred_ref, attn_ref, masked_ref):
    featT = jnp.dot(fcw_ref[...], poolT_ref[...],
                    preferred_element_type=jnp.float32) + fcb_ref[...]
    eps = jnp.float32(jnp.finfo(jnp.float32).eps)
    ms = jnp.mean(featT * featT, axis=0, keepdims=True)
    featT = featT * jax.lax.rsqrt(ms + eps) * g_ref[...]
    x_vT = jnp.dot(wvw_ref[...], featT,
                   preferred_element_type=jnp.float32) + wvb_ref[...]
    a1T = jnp.tanh(jnp.dot(w1w_ref[...], featT,
                           preferred_element_type=jnp.float32) + w1b_ref[...])
    raw_row = jnp.dot(vw_ref[...], a1T,
                      preferred_element_type=jnp.float32) + vb_ref[...]
    pred_row = jnp.dot(pw_ref[...], x_vT,
                       preferred_element_type=jnp.float32) * 100.0

    s = jnp.concatenate(
        [raw_row[:, 16 * c:16 * (c + 1)] for c in range(4)], axis=0)
    t = jax.lax.broadcasted_iota(jnp.int32, (4, 16), 1)
    neg = jnp.full((4, 16), jnp.float32(-3.0e38))
    processed = jnp.zeros((4, 16), dtype=jnp.float32)
    mask = jnp.ones((4, 16), dtype=jnp.float32)
    keep = jnp.ones((4, 16), dtype=jnp.float32)
    for _ in range(16):
        cand = jnp.where(processed > 0.5, neg, s)
        mx = jnp.max(cand, axis=-1, keepdims=True)
        ii = jnp.min(jnp.where(cand == mx, t, 99), axis=-1, keepdims=True)
        is_i = jnp.where(t == ii, 1.0, 0.0)
        cond = jnp.max(is_i * mask, axis=-1, keepdims=True)
        window = jnp.abs(t - ii) <= 1
        mask = jnp.where(cond > 0.5, jnp.where(window, is_i, mask), mask)
        keep = jnp.where(t == ii, jnp.where(cond > 0.5, keep, 0.0), keep)
        processed = jnp.maximum(processed, is_i)

    keep_row = jnp.concatenate(
        [keep[c:c + 1, :] for c in range(4)], axis=1)

    rawm = jnp.where(keep_row > 0.5, raw_row, _F32_MIN)
    fp = 1.0 / (1.0 + jnp.exp(-rawm))
    fp = jnp.where(jnp.abs(fp) < jnp.inf, fp, 0.0)

    v = jnp.maximum(fp, 0.0)
    i0 = jax.lax.broadcasted_iota(jnp.int32, (64, 64), 0)
    i1 = jax.lax.broadcasted_iota(jnp.int32, (64, 64), 1)
    eye = (i0 == i1).astype(jnp.float32)
    vcol = jax.lax.dot_general(eye, v, (((1,), (1,)), ((), ())),
                               preferred_element_type=jnp.float32)
    gi = jax.lax.broadcasted_iota(jnp.int32, (1, 64), 1)
    gj = jax.lax.broadcasted_iota(jnp.int32, (64, 1), 0)
    higher = vcol > v
    tie = vcol == v
    vj_b = jnp.broadcast_to(vcol, (64, 64))
    csum_part = jnp.where(higher, vj_b, jnp.where(tie & (gj <= gi), vj_b, 0.0))
    csum = jnp.sum(csum_part, axis=0, keepdims=True)
    rank_part = jnp.where(higher, 1.0, jnp.where(tie & (gj < gi), 1.0, 0.0))
    rank = jnp.sum(rank_part, axis=0, keepdims=True)
    total = jnp.sum(v, axis=1, keepdims=True)
    keep_tp = jnp.where(csum / (total + 1e-08) <= 0.7, 1.0,
                        jnp.where(rank < 3.0, 1.0, 0.0))

    masked = fp * keep_tp
    ssum = jnp.sum(masked, axis=1, keepdims=True)
    attn = masked / (ssum + 1e-08)
    attn = jnp.where(ssum <= 0.0, jnp.full((1, 64), jnp.float32(1.0 / 64.0)), attn)

    out0_ref[...] = jnp.sum(attn * pred_row, axis=1, keepdims=True)
    pred_ref[...] = pred_row
    attn_ref[...] = attn
    masked_ref[...] = masked


def _head(poolT, fc_w, fc_b, rms_g, WV_w, WV_b, W1_w, W1_b, V_w, V_b, P_w):
    out_shapes = (
        jax.ShapeDtypeStruct((1, 1), jnp.float32),
        jax.ShapeDtypeStruct((1, 64), jnp.float32),
        jax.ShapeDtypeStruct((1, 64), jnp.float32),
        jax.ShapeDtypeStruct((1, 64), jnp.float32),
    )
    return pl.pallas_call(_head_body, out_shape=out_shapes)(
        poolT, fc_w, fc_b.reshape(2048, 1), rms_g.reshape(2048, 1),
        WV_w, WV_b.reshape(128, 1), W1_w, W1_b.reshape(64, 1),
        V_w, V_b.reshape(1, 1), P_w)


def _conv1_s2d(x):
    xi = x.reshape(64, 3, 224, 224)
    xp = jnp.pad(xi, ((0, 0), (0, 0), (1, 3), (1, 3)))
    xa = xp.reshape(64, 3, 57, 4, 57, 4).transpose(0, 2, 4, 3, 5, 1)
    return xa.reshape(64, 3249, 48)


def _conv1_weight(conv1_w):
    wp = jnp.pad(conv1_w, ((0, 0), (0, 0), (0, 1), (0, 1)))
    wp = wp.reshape(64, 3, 2, 4, 2, 4)
    return wp.transpose(2, 3, 5, 1, 4, 0).reshape(2, 48, 128)


def _conv2_weights(conv2_w):
    wt = conv2_w.transpose(2, 3, 1, 0)
    wa = jnp.concatenate(
        [wt[kh, kw] for kh in range(3) for kw in (0, 1)], axis=0)
    wb = jnp.concatenate([wt[kh, 2] for kh in range(3)], axis=0)
    return wa, wb


def kernel(x, conv1_w, conv2_w, fc_w, fc_b, rms_g, WV_w, WV_b, W1_w, W1_b, V_w, V_b, P_w):
    hee, heo, hoe, hoo = _conv1(_conv1_s2d(x), _conv1_weight(conv1_w))
    wa, wb = _conv2_weights(conv2_w)
    pooled = _conv2_pool(hee, heo, hoe, hoo, wa, wb)
    out0, pred_row, attn_row, masked_row = _head(
        pooled.T, fc_w, fc_b, rms_g, WV_w, WV_b, W1_w, W1_b, V_w, V_b, P_w)
    pred_by_frame = pred_row.reshape(1, 4, 16, 1)
    attn = attn_row.reshape(1, 4, 16, 1)
    masked = masked_row.reshape(1, 4, 16, 1)
    return out0, pred_by_frame, attn, masked
